# baseline (speedup 1.0000x reference)
import functools

import jax
import jax.numpy as jnp
from jax import lax
from jax.experimental import pallas as pl
from jax.experimental.pallas import tpu as pltpu
from jax.experimental.pallas import tpu_sc as plsc

N = 100000
EMB = 64
LW = 2 * EMB
VOCAB = 1000000
CHUNK = 128
NC = 2
NS = 16
NW = NC * NS
CH_PER_W = 25
PER_W = CH_PER_W * CHUNK
NPAD = PER_W * NW
NCH = NPAD // CHUNK
NBUF = 5
NGRP = CH_PER_W // NBUF

PREP_BLK = 8192


def _prep_body(t_ref, m_ref, e_ref, o_ref):
    t = t_ref[...]
    o_ref[...] = (jnp.dot(t, m_ref[...], preferred_element_type=jnp.float32)
                  + e_ref[...])


def _prep(table, m, e):
    return pl.pallas_call(
        _prep_body,
        grid=(VOCAB // PREP_BLK,),
        in_specs=[
            pl.BlockSpec((PREP_BLK, EMB), lambda i: (i, 0)),
            pl.BlockSpec((EMB, LW), lambda i: (0, 0)),
            pl.BlockSpec((1, LW), lambda i: (0, 0)),
        ],
        out_specs=pl.BlockSpec((PREP_BLK, LW), lambda i: (i, 0)),
        out_shape=jax.ShapeDtypeStruct((VOCAB, LW), jnp.float32),
    )(table, m, e)


@functools.cache
def _sc_gather():
    mesh = plsc.VectorSubcoreMesh(
        core_axis_name="c", subcore_axis_name="s",
        num_cores=NC, num_subcores=NS)

    @functools.partial(
        pl.kernel,
        out_type=jax.ShapeDtypeStruct((NCH, CHUNK, LW), jnp.float32),
        mesh=mesh,
        scratch_types=[
            pltpu.VMEM((PER_W,), jnp.int32),
            pltpu.VMEM((NBUF, CHUNK, LW), jnp.float32),
            pltpu.SemaphoreType.DMA,
        ],
        compiler_params=pltpu.CompilerParams(use_tc_tiling_on_sc=True),
    )
    def gather(tbl, idx_hbm, out, idx_v, bufs, sem):
        wid = lax.axis_index("s") * NC + lax.axis_index("c")
        base = wid * PER_W
        pltpu.sync_copy(idx_hbm.at[pl.ds(base, PER_W)], idx_v)

        for b in range(NBUF):
            pltpu.async_copy(
                tbl.at[idx_v.at[pl.ds(b * CHUNK, CHUNK)]],
                bufs.at[b], sem)

        def grp(g, carry):
            for b in range(NBUF):
                c = g * NBUF + b
                pltpu.make_async_copy(
                    tbl.at[idx_v.at[pl.ds(0, CHUNK)]],
                    bufs.at[b], sem).wait()
                pltpu.sync_copy(bufs.at[b], out.at[wid * CH_PER_W + c])

                @pl.when(g < NGRP - 1)
                def _():
                    pltpu.async_copy(
                        tbl.at[idx_v.at[pl.ds((c + NBUF) * CHUNK, CHUNK)]],
                        bufs.at[b], sem)
            return carry

        lax.fori_loop(0, NGRP, grp, 0)

    return gather


RED_BLK = 4096


def _red_body(us_ref, zs_ref, o_ref):
    o_ref[...] = jnp.sum(us_ref[...] * zs_ref[...], axis=1)


def kernel(U, V, B, us_ind, vs_ind):
    us_pad = jnp.concatenate(
        [us_ind.astype(jnp.int32), jnp.zeros((NPAD - N,), jnp.int32)])
    vs_pad = jnp.concatenate(
        [vs_ind.astype(jnp.int32), jnp.zeros((NPAD - N,), jnp.int32)])

    b00 = B[:EMB, :EMB]
    mu = (jnp.zeros((EMB, LW), jnp.float32)
          .at[:, :EMB].set(jnp.eye(EMB, dtype=jnp.float32))
          .at[:, EMB].set(B[:EMB, EMB]))
    eu = jnp.zeros((1, LW), jnp.float32).at[0, EMB + 1].set(1.0)
    mz = (jnp.zeros((EMB, LW), jnp.float32)
          .at[:, :EMB].set(b00.T)
          .at[:, EMB + 1].set(B[EMB, :EMB]))
    ez = jnp.zeros((1, LW), jnp.float32).at[0, EMB].set(1.0)

    u128 = _prep(U, mu, eu)
    z128 = _prep(V, mz, ez)

    us3 = _sc_gather()(u128, us_pad)
    zs3 = _sc_gather()(z128, vs_pad)
    us2 = us3.reshape(NPAD, LW)
    zs2 = zs3.reshape(NPAD, LW)

    act_pad = pl.pallas_call(
        _red_body,
        grid=(NPAD // RED_BLK,),
        in_specs=[
            pl.BlockSpec((RED_BLK, LW), lambda i: (i, 0)),
            pl.BlockSpec((RED_BLK, LW), lambda i: (i, 0)),
        ],
        out_specs=pl.BlockSpec((RED_BLK,), lambda i: (i,)),
        out_shape=jax.ShapeDtypeStruct((NPAD,), jnp.float32),
    )(us2, zs2)

    return act_pad[:N] + B[EMB, EMB]


# speedup vs baseline: 1.2380x; 1.2380x over previous
import functools

import jax
import jax.numpy as jnp
from jax import lax
from jax.experimental import pallas as pl
from jax.experimental.pallas import tpu as pltpu
from jax.experimental.pallas import tpu_sc as plsc

N = 100000
EMB = 64
CHUNK = 128
NC = 2
NS = 16
NW = NC * NS
CH_PER_W = 25
PER_W = CH_PER_W * CHUNK
NPAD = PER_W * NW
NPAIR = NPAD // 2
NBUF = 5
NGRP = CH_PER_W // NBUF


@functools.cache
def _sc_gather():
    mesh = plsc.VectorSubcoreMesh(
        core_axis_name="c", subcore_axis_name="s",
        num_cores=NC, num_subcores=NS)

    @functools.partial(
        pl.kernel,
        out_type=[
            jax.ShapeDtypeStruct((NPAD // CHUNK, CHUNK, EMB), jnp.float32),
            jax.ShapeDtypeStruct((NPAD // CHUNK, CHUNK, EMB), jnp.float32),
        ],
        mesh=mesh,
        scratch_types=[
            pltpu.VMEM((PER_W,), jnp.int32),
            pltpu.VMEM((PER_W,), jnp.int32),
            pltpu.VMEM((NBUF, CHUNK, EMB), jnp.float32),
            pltpu.SemaphoreType.DMA,
        ],
        compiler_params=pltpu.CompilerParams(use_tc_tiling_on_sc=False),
    )
    def gather(u_hbm, v_hbm, us_idx, vs_idx, us_out, vs_out,
               uidx_v, vidx_v, bufs, sem):
        wid = lax.axis_index("s") * NC + lax.axis_index("c")
        base = wid * PER_W
        pltpu.sync_copy(us_idx.at[pl.ds(base, PER_W)], uidx_v)
        pltpu.sync_copy(vs_idx.at[pl.ds(base, PER_W)], vidx_v)

        def phase(tbl, idx_v, out):
            for b in range(NBUF):
                pltpu.async_copy(
                    tbl.at[idx_v.at[pl.ds(b * CHUNK, CHUNK)]],
                    bufs.at[b], sem)

            def grp(g, carry):
                for b in range(NBUF):
                    c = g * NBUF + b
                    pltpu.make_async_copy(
                        tbl.at[idx_v.at[pl.ds(0, CHUNK)]],
                        bufs.at[b], sem).wait()
                    pltpu.sync_copy(bufs.at[b], out.at[wid * CH_PER_W + c])

                    @pl.when(g < NGRP - 1)
                    def _():
                        pltpu.async_copy(
                            tbl.at[idx_v.at[pl.ds((c + NBUF) * CHUNK, CHUNK)]],
                            bufs.at[b], sem)
                return carry

            lax.fori_loop(0, NGRP, grp, 0)

        phase(u_hbm, uidx_v, us_out)
        phase(v_hbm, vidx_v, vs_out)

    return gather


BLK2 = 2048


def _tc_body(us_ref, vs_ref, bm_ref, bvu_ref, oe_ref, oo_ref):
    u = us_ref[...]
    v = vs_ref[...]
    t = jnp.dot(u, bm_ref[...], preferred_element_type=jnp.float32)
    bvu = bvu_ref[...]
    prod = (t + bvu[0:1, :]) * v + u * bvu[1:2, :]
    oe_ref[...] = jnp.sum(prod[:, :EMB], axis=1)
    oo_ref[...] = jnp.sum(prod[:, EMB:], axis=1)


def kernel(U, V, B, us_ind, vs_ind):
    us_pad = jnp.concatenate(
        [us_ind.astype(jnp.int32), jnp.zeros((NPAD - N,), jnp.int32)])
    vs_pad = jnp.concatenate(
        [vs_ind.astype(jnp.int32), jnp.zeros((NPAD - N,), jnp.int32)])

    us3, vs3 = _sc_gather()(U, V, us_pad, vs_pad)
    us2 = us3.reshape(NPAIR, 2 * EMB)
    vs2 = vs3.reshape(NPAIR, 2 * EMB)

    b00 = B[:EMB, :EMB]
    bm = (jnp.zeros((2 * EMB, 2 * EMB), jnp.float32)
          .at[:EMB, :EMB].set(b00)
          .at[EMB:, EMB:].set(b00))
    bvu = jnp.concatenate([
        jnp.tile(B[EMB, :EMB], 2)[None, :],
        jnp.tile(B[:EMB, EMB], 2)[None, :],
    ], axis=0)

    oe, oo = pl.pallas_call(
        _tc_body,
        grid=(NPAIR // BLK2,),
        in_specs=[
            pl.BlockSpec((BLK2, 2 * EMB), lambda i: (i, 0)),
            pl.BlockSpec((BLK2, 2 * EMB), lambda i: (i, 0)),
            pl.BlockSpec((2 * EMB, 2 * EMB), lambda i: (0, 0)),
            pl.BlockSpec((2, 2 * EMB), lambda i: (0, 0)),
        ],
        out_specs=[
            pl.BlockSpec((BLK2,), lambda i: (i,)),
            pl.BlockSpec((BLK2,), lambda i: (i,)),
        ],
        out_shape=[
            jax.ShapeDtypeStruct((NPAIR,), jnp.float32),
            jax.ShapeDtypeStruct((NPAIR,), jnp.float32),
        ],
    )(us2, vs2, bm, bvu)

    act = jnp.stack([oe, oo], axis=1).reshape(NPAD)[:N] + B[EMB, EMB]
    return act


# speedup vs baseline: 1.2927x; 1.0442x over previous
import functools

import jax
import jax.numpy as jnp
from jax import lax
from jax.experimental import pallas as pl
from jax.experimental.pallas import tpu as pltpu
from jax.experimental.pallas import tpu_sc as plsc

N = 100000
EMB = 64
CHUNK = 128
NC = 2
NS = 16
NW = NC * NS
CH_PER_W = 25
PER_W = CH_PER_W * CHUNK
NPAD = PER_W * NW
NPAIR = NPAD // 2
NBUF = 5
NGRP = CH_PER_W // NBUF


@functools.cache
def _sc_gather():
    mesh = plsc.VectorSubcoreMesh(
        core_axis_name="c", subcore_axis_name="s",
        num_cores=NC, num_subcores=NS)

    @functools.partial(
        pl.kernel,
        out_type=jax.ShapeDtypeStruct((NPAD // CHUNK, CHUNK, EMB), jnp.float32),
        mesh=mesh,
        scratch_types=[
            pltpu.VMEM((PER_W,), jnp.int32),
            pltpu.VMEM((NBUF, CHUNK, EMB), jnp.float32),
            pltpu.SemaphoreType.DMA,
        ],
        compiler_params=pltpu.CompilerParams(use_tc_tiling_on_sc=False),
    )
    def gather(tbl, idx_hbm, out, idx_v, bufs, sem):
        wid = lax.axis_index("s") * NC + lax.axis_index("c")
        base = wid * PER_W
        pltpu.sync_copy(idx_hbm.at[pl.ds(base, PER_W)], idx_v)

        for b in range(NBUF):
            pltpu.async_copy(
                tbl.at[idx_v.at[pl.ds(b * CHUNK, CHUNK)]],
                bufs.at[b], sem)

        def grp(g, carry):
            for b in range(NBUF):
                c = g * NBUF + b
                pltpu.make_async_copy(
                    tbl.at[idx_v.at[pl.ds(0, CHUNK)]],
                    bufs.at[b], sem).wait()
                pltpu.sync_copy(bufs.at[b], out.at[wid * CH_PER_W + c])

                @pl.when(g < NGRP - 1)
                def _():
                    pltpu.async_copy(
                        tbl.at[idx_v.at[pl.ds((c + NBUF) * CHUNK, CHUNK)]],
                        bufs.at[b], sem)
            return carry

        lax.fori_loop(0, NGRP, grp, 0)

    return gather


BLK2 = 2048


def _tc_body(us_ref, vs_ref, bm_ref, bvu_ref, oe_ref, oo_ref):
    u = us_ref[...]
    v = vs_ref[...]
    t = jnp.dot(u, bm_ref[...], preferred_element_type=jnp.float32)
    bvu = bvu_ref[...]
    prod = (t + bvu[0:1, :]) * v + u * bvu[1:2, :]
    oe_ref[...] = jnp.sum(prod[:, :EMB], axis=1)
    oo_ref[...] = jnp.sum(prod[:, EMB:], axis=1)


def kernel(U, V, B, us_ind, vs_ind):
    us_pad = jnp.concatenate(
        [us_ind.astype(jnp.int32), jnp.zeros((NPAD - N,), jnp.int32)])
    vs_pad = jnp.concatenate(
        [vs_ind.astype(jnp.int32), jnp.zeros((NPAD - N,), jnp.int32)])

    us3 = _sc_gather()(U, us_pad)
    vs3 = _sc_gather()(V, vs_pad)
    us2 = us3.reshape(NPAIR, 2 * EMB)
    vs2 = vs3.reshape(NPAIR, 2 * EMB)

    b00 = B[:EMB, :EMB]
    bm = (jnp.zeros((2 * EMB, 2 * EMB), jnp.float32)
          .at[:EMB, :EMB].set(b00)
          .at[EMB:, EMB:].set(b00))
    bvu = jnp.concatenate([
        jnp.tile(B[EMB, :EMB], 2)[None, :],
        jnp.tile(B[:EMB, EMB], 2)[None, :],
    ], axis=0)

    oe, oo = pl.pallas_call(
        _tc_body,
        grid=(NPAIR // BLK2,),
        in_specs=[
            pl.BlockSpec((BLK2, 2 * EMB), lambda i: (i, 0)),
            pl.BlockSpec((BLK2, 2 * EMB), lambda i: (i, 0)),
            pl.BlockSpec((2 * EMB, 2 * EMB), lambda i: (0, 0)),
            pl.BlockSpec((2, 2 * EMB), lambda i: (0, 0)),
        ],
        out_specs=[
            pl.BlockSpec((BLK2,), lambda i: (i,)),
            pl.BlockSpec((BLK2,), lambda i: (i,)),
        ],
        out_shape=[
            jax.ShapeDtypeStruct((NPAIR,), jnp.float32),
            jax.ShapeDtypeStruct((NPAIR,), jnp.float32),
        ],
    )(us2, vs2, bm, bvu)

    act = jnp.stack([oe, oo], axis=1).reshape(NPAD)[:N] + B[EMB, EMB]
    return act
